# baseline (speedup 1.0000x reference)
import functools

import jax
import jax.numpy as jnp
from jax.experimental import pallas as pl
from jax.experimental.pallas import tpu as pltpu

_Q_BLK = 784
_K_PAD = 10240
_CK = 2048
_N_CHUNK = _K_PAD // _CK
_PAD_VAL = 100.0


def _qsq_body(q_ref, qsq_ref):
    q = q_ref[...]
    qsq_ref[...] = jnp.sum(q * q, axis=1, keepdims=True)


def _ksq_body(kt_ref, ksq_ref):
    kt = kt_ref[...]
    ksq_ref[...] = jnp.sum(kt * kt, axis=0, keepdims=True)


def _knn_body(qb_ref, qsq_ref, kt_ref, ksq_ref, map_ref, max_ref):
    qb = qb_ref[...]

    def chunk(c, acc):
        prod = jax.lax.dot_general(
            qb, kt_ref[:, pl.ds(c * _CK, _CK)],
            (((1,), (0,)), ((), ())),
            preferred_element_type=jnp.float32)
        t = prod + ksq_ref[:, pl.ds(c * _CK, _CK)]
        for s in range(_CK // 128):
            acc = jnp.minimum(acc, t[:, s * 128:(s + 1) * 128])
        return acc

    acc = jax.lax.fori_loop(
        0, _N_CHUNK, chunk,
        jnp.full((_Q_BLK, 128), jnp.inf, jnp.float32))
    mins = jnp.min(acc, axis=1, keepdims=True)
    dist = jnp.sqrt(jnp.maximum(mins + qsq_ref[...], 1e-12))
    map_ref[0, :, :] = dist
    max_ref[0, :, :] = jnp.max(dist, axis=(0, 1), keepdims=True)


@functools.partial(jax.jit, static_argnames=())
def kernel(queries, keys):
    n_img = queries.shape[0] // _Q_BLK
    d = queries.shape[1]
    kt = jnp.pad(keys.T, ((0, 0), (0, _K_PAD - keys.shape[0])),
                 constant_values=_PAD_VAL)
    ktb = kt.astype(jnp.bfloat16)
    qb = (-2.0 * queries).astype(jnp.bfloat16)

    qsq = pl.pallas_call(
        _qsq_body,
        grid=(n_img,),
        in_specs=[pl.BlockSpec((_Q_BLK, d), lambda i: (i, 0))],
        out_specs=pl.BlockSpec((_Q_BLK, 1), lambda i: (i, 0)),
        out_shape=jax.ShapeDtypeStruct((n_img * _Q_BLK, 1), jnp.float32),
        compiler_params=pltpu.CompilerParams(
            dimension_semantics=("parallel",)),
    )(queries)

    ksq = pl.pallas_call(
        _ksq_body,
        grid=(_N_CHUNK,),
        in_specs=[pl.BlockSpec((d, _CK), lambda i: (0, i))],
        out_specs=pl.BlockSpec((1, _CK), lambda i: (0, i)),
        out_shape=jax.ShapeDtypeStruct((1, _K_PAD), jnp.float32),
        compiler_params=pltpu.CompilerParams(
            dimension_semantics=("parallel",)),
    )(kt)

    amap, amax = pl.pallas_call(
        _knn_body,
        grid=(n_img,),
        in_specs=[
            pl.BlockSpec((_Q_BLK, d), lambda i: (i, 0)),
            pl.BlockSpec((_Q_BLK, 1), lambda i: (i, 0)),
            pl.BlockSpec(ktb.shape, lambda i: (0, 0)),
            pl.BlockSpec((1, _K_PAD), lambda i: (0, 0)),
        ],
        out_specs=[
            pl.BlockSpec((1, _Q_BLK, 1), lambda i: (i, 0, 0)),
            pl.BlockSpec((1, 1, 1), lambda i: (i, 0, 0)),
        ],
        out_shape=[
            jax.ShapeDtypeStruct((n_img, _Q_BLK, 1), jnp.float32),
            jax.ShapeDtypeStruct((n_img, 1, 1), jnp.float32),
        ],
        compiler_params=pltpu.CompilerParams(
            dimension_semantics=("parallel",)),
    )(qb, qsq, ktb, ksq)
    return amax.reshape(n_img), amap.reshape(n_img, 28, 28)


# speedup vs baseline: 1.2898x; 1.2898x over previous
import functools

import jax
import jax.numpy as jnp
from jax.experimental import pallas as pl
from jax.experimental.pallas import tpu as pltpu

_Q_BLK = 784
_K_PAD = 10240
_CK = 2048
_N_CHUNK = _K_PAD // _CK
_D_AUG = 512
_PAD_VAL = 100.0


def _kprep_body(k_ref, kb_ref):
    k = k_ref[...]
    ksq = jnp.sum(k * k, axis=1, keepdims=True)
    hi = ksq.astype(jnp.bfloat16)
    lo = (ksq - hi.astype(jnp.float32)).astype(jnp.bfloat16)
    zeros = jnp.zeros((k.shape[0], _D_AUG - k.shape[1] - 2), jnp.bfloat16)
    kb_ref[...] = jnp.concatenate(
        [k.astype(jnp.bfloat16), hi, lo, zeros], axis=1)


def _qprep_body(q_ref, qb_ref, qsq_ref):
    q = q_ref[...]
    qsq_ref[...] = jnp.sum(q * q, axis=1, keepdims=True)
    ones = jnp.ones((q.shape[0], 2), jnp.bfloat16)
    zeros = jnp.zeros((q.shape[0], _D_AUG - q.shape[1] - 2), jnp.bfloat16)
    qb_ref[...] = jnp.concatenate(
        [(-2.0 * q).astype(jnp.bfloat16), ones, zeros], axis=1)


def _knn_body(qb_ref, qsq_ref, kb_ref, map_ref, max_ref):
    qb = qb_ref[...]

    acc = jnp.full((_Q_BLK, 128), jnp.inf, jnp.float32)
    for c in range(_N_CHUNK):
        t = jax.lax.dot_general(
            qb, kb_ref[pl.ds(c * _CK, _CK), :],
            (((1,), (1,)), ((), ())),
            preferred_element_type=jnp.float32)
        for s in range(_CK // 128):
            acc = jnp.minimum(acc, t[:, s * 128:(s + 1) * 128])

    mins = jnp.min(acc, axis=1, keepdims=True)
    dist = jnp.sqrt(jnp.maximum(mins + qsq_ref[...], 1e-12))
    map_ref[0, :, :] = dist
    max_ref[0, :, :] = jnp.max(dist, axis=(0, 1), keepdims=True)


@functools.partial(jax.jit, static_argnames=())
def kernel(queries, keys):
    n_img = queries.shape[0] // _Q_BLK
    d = queries.shape[1]
    kp = jnp.pad(keys, ((0, _K_PAD - keys.shape[0]), (0, 0)),
                 constant_values=_PAD_VAL)

    kb = pl.pallas_call(
        _kprep_body,
        grid=(_N_CHUNK,),
        in_specs=[pl.BlockSpec((_CK, d), lambda i: (i, 0))],
        out_specs=pl.BlockSpec((_CK, _D_AUG), lambda i: (i, 0)),
        out_shape=jax.ShapeDtypeStruct((_K_PAD, _D_AUG), jnp.bfloat16),
    )(kp)

    qb, qsq = pl.pallas_call(
        _qprep_body,
        grid=(n_img,),
        in_specs=[pl.BlockSpec((_Q_BLK, d), lambda i: (i, 0))],
        out_specs=[
            pl.BlockSpec((_Q_BLK, _D_AUG), lambda i: (i, 0)),
            pl.BlockSpec((_Q_BLK, 1), lambda i: (i, 0)),
        ],
        out_shape=[
            jax.ShapeDtypeStruct((n_img * _Q_BLK, _D_AUG), jnp.bfloat16),
            jax.ShapeDtypeStruct((n_img * _Q_BLK, 1), jnp.float32),
        ],
    )(queries)

    amap, amax = pl.pallas_call(
        _knn_body,
        grid=(n_img,),
        in_specs=[
            pl.BlockSpec((_Q_BLK, _D_AUG), lambda i: (i, 0)),
            pl.BlockSpec((_Q_BLK, 1), lambda i: (i, 0)),
            pl.BlockSpec((_K_PAD, _D_AUG), lambda i: (0, 0)),
        ],
        out_specs=[
            pl.BlockSpec((1, _Q_BLK, 1), lambda i: (i, 0, 0)),
            pl.BlockSpec((1, 1, 1), lambda i: (i, 0, 0)),
        ],
        out_shape=[
            jax.ShapeDtypeStruct((n_img, _Q_BLK, 1), jnp.float32),
            jax.ShapeDtypeStruct((n_img, 1, 1), jnp.float32),
        ],
    )(qb, qsq, kb)
    return amax.reshape(n_img), amap.reshape(n_img, 28, 28)


# speedup vs baseline: 1.5230x; 1.1808x over previous
import functools

import jax
import jax.numpy as jnp
from jax.experimental import pallas as pl
from jax.experimental.pallas import tpu as pltpu

_Q_BLK = 784
_K_BANK = 10000
_K_PAD = 10240
_CK = 2048
_N_CHUNK = _K_PAD // _CK
_D_AUG = 512
_PAD_VAL = 100.0


def _kprep_body(k_ref, kb_ref):
    i = pl.program_id(0)
    k = k_ref[...]
    row = jax.lax.broadcasted_iota(jnp.int32, k.shape, 0) + i * _CK
    k = jnp.where(row < _K_BANK, k, _PAD_VAL)
    ksq = jnp.sum(k * k, axis=1, keepdims=True)
    hi = ksq.astype(jnp.bfloat16)
    lo = (ksq - hi.astype(jnp.float32)).astype(jnp.bfloat16)
    zeros = jnp.zeros((k.shape[0], _D_AUG - k.shape[1] - 2), jnp.bfloat16)
    kb_ref[...] = jnp.concatenate(
        [k.astype(jnp.bfloat16), hi, lo, zeros], axis=1)


def _knn_body(q_ref, kb_ref, map_ref, max_ref):
    q = q_ref[...]
    qsq = jnp.sum(q * q, axis=1, keepdims=True)
    ones = jnp.ones((q.shape[0], 2), jnp.bfloat16)
    zeros = jnp.zeros((q.shape[0], _D_AUG - q.shape[1] - 2), jnp.bfloat16)
    qb = jnp.concatenate(
        [(-2.0 * q).astype(jnp.bfloat16), ones, zeros], axis=1)

    acc = jnp.full((_Q_BLK, 128), jnp.inf, jnp.float32)
    for c in range(_N_CHUNK):
        t = jax.lax.dot_general(
            qb, kb_ref[pl.ds(c * _CK, _CK), :],
            (((1,), (1,)), ((), ())),
            preferred_element_type=jnp.float32)
        for s in range(_CK // 128):
            acc = jnp.minimum(acc, t[:, s * 128:(s + 1) * 128])

    mins = jnp.min(acc, axis=1, keepdims=True)
    dist = jnp.sqrt(jnp.maximum(mins + qsq, 1e-12))
    map_ref[0, :, :] = dist
    max_ref[0, :, :] = jnp.max(dist, axis=(0, 1), keepdims=True)


@functools.partial(jax.jit, static_argnames=())
def kernel(queries, keys):
    n_img = queries.shape[0] // _Q_BLK
    d = queries.shape[1]

    kb = pl.pallas_call(
        _kprep_body,
        grid=(_N_CHUNK,),
        in_specs=[pl.BlockSpec((_CK, d), lambda i: (i, 0))],
        out_specs=pl.BlockSpec((_CK, _D_AUG), lambda i: (i, 0)),
        out_shape=jax.ShapeDtypeStruct((_K_PAD, _D_AUG), jnp.bfloat16),
    )(keys)

    amap, amax = pl.pallas_call(
        _knn_body,
        grid=(n_img,),
        in_specs=[
            pl.BlockSpec((_Q_BLK, d), lambda i: (i, 0)),
            pl.BlockSpec((_K_PAD, _D_AUG), lambda i: (0, 0)),
        ],
        out_specs=[
            pl.BlockSpec((1, _Q_BLK, 1), lambda i: (i, 0, 0)),
            pl.BlockSpec((1, 1, 1), lambda i: (i, 0, 0)),
        ],
        out_shape=[
            jax.ShapeDtypeStruct((n_img, _Q_BLK, 1), jnp.float32),
            jax.ShapeDtypeStruct((n_img, 1, 1), jnp.float32),
        ],
    )(queries, kb)
    return amax.reshape(n_img), amap.reshape(n_img, 28, 28)


# speedup vs baseline: 1.5780x; 1.0362x over previous
import functools

import jax
import jax.numpy as jnp
from jax.experimental import pallas as pl
from jax.experimental.pallas import tpu as pltpu

_Q_BLK = 1568
_IMG = 784
_K_BANK = 10000
_K_PAD = 10240
_CK = 2048
_N_CHUNK = _K_PAD // _CK
_D_AUG = 512
_PAD_VAL = 100.0


def _kprep_body(k_ref, kb_ref):
    i = pl.program_id(0)
    k = k_ref[...]
    row = jax.lax.broadcasted_iota(jnp.int32, k.shape, 0) + i * _CK
    k = jnp.where(row < _K_BANK, k, _PAD_VAL)
    ksq = jnp.sum(k * k, axis=1, keepdims=True)
    hi = ksq.astype(jnp.bfloat16)
    lo = (ksq - hi.astype(jnp.float32)).astype(jnp.bfloat16)
    zeros = jnp.zeros((k.shape[0], _D_AUG - k.shape[1] - 2), jnp.bfloat16)
    kb_ref[...] = jnp.concatenate(
        [k.astype(jnp.bfloat16), hi, lo, zeros], axis=1)


def _knn_body(q_ref, kb_ref, map_ref, max_ref):
    q = q_ref[...]
    qsq = jnp.sum(q * q, axis=1, keepdims=True)
    ones = jnp.ones((q.shape[0], 2), jnp.bfloat16)
    zeros = jnp.zeros((q.shape[0], _D_AUG - q.shape[1] - 2), jnp.bfloat16)
    qb = jnp.concatenate(
        [(-2.0 * q).astype(jnp.bfloat16), ones, zeros], axis=1)

    acc = jnp.full((_Q_BLK, 128), jnp.inf, jnp.float32)
    for c in range(_N_CHUNK):
        t = jax.lax.dot_general(
            qb, kb_ref[pl.ds(c * _CK, _CK), :],
            (((1,), (1,)), ((), ())),
            preferred_element_type=jnp.float32)
        for s in range(_CK // 128):
            acc = jnp.minimum(acc, t[:, s * 128:(s + 1) * 128])

    mins = jnp.min(acc, axis=1, keepdims=True)
    dist = jnp.sqrt(jnp.maximum(mins + qsq, 1e-12))
    map_ref[0, :, :] = dist
    m0 = jnp.max(dist[:_IMG], axis=(0, 1), keepdims=True)
    m1 = jnp.max(dist[_IMG:], axis=(0, 1), keepdims=True)
    max_ref[0, :, :] = jnp.concatenate([m0, m1], axis=0)


@functools.partial(jax.jit, static_argnames=())
def kernel(queries, keys):
    n_blk = queries.shape[0] // _Q_BLK
    n_img = queries.shape[0] // _IMG
    d = queries.shape[1]

    kb = pl.pallas_call(
        _kprep_body,
        grid=(_N_CHUNK,),
        in_specs=[pl.BlockSpec((_CK, d), lambda i: (i, 0))],
        out_specs=pl.BlockSpec((_CK, _D_AUG), lambda i: (i, 0)),
        out_shape=jax.ShapeDtypeStruct((_K_PAD, _D_AUG), jnp.bfloat16),
    )(keys)

    amap, amax = pl.pallas_call(
        _knn_body,
        grid=(n_blk,),
        in_specs=[
            pl.BlockSpec((_Q_BLK, d), lambda i: (i, 0)),
            pl.BlockSpec((_K_PAD, _D_AUG), lambda i: (0, 0)),
        ],
        out_specs=[
            pl.BlockSpec((1, _Q_BLK, 1), lambda i: (i, 0, 0)),
            pl.BlockSpec((1, 2, 1), lambda i: (i, 0, 0)),
        ],
        out_shape=[
            jax.ShapeDtypeStruct((n_blk, _Q_BLK, 1), jnp.float32),
            jax.ShapeDtypeStruct((n_blk, 2, 1), jnp.float32),
        ],
    )(queries, kb)
    return amax.reshape(n_img), amap.reshape(n_img, 28, 28)
